# baseline (speedup 1.0000x reference)
import functools

import numpy as np
import jax
import jax.numpy as jnp
from jax import lax
from jax.experimental import pallas as pl
from jax.experimental.pallas import tpu as pltpu
from jax.experimental.pallas import tpu_sc as plsc

N = 10000
E = 160000
F = 128
S = 16
DEDGE = 4
COUT = 16
BLK = 256
NW = 32
CH = 128
G = 4
NROWS = 10240
NR16 = NROWS // 16
ROWW = F + S
ETROWS = 160256
ETW = 16
XROWS = NROWS


def _round_up(a, b):
    return (a + b - 1) // b * b


def _corner_table(K):
    nc = K - 1
    C = nc ** DEDGE
    offs = np.array([[(s >> d) & 1 for d in range(DEDGE)] for s in range(S)],
                    dtype=np.int32)
    strides = np.array([K ** d for d in range(DEDGE)], dtype=np.int64)
    wsel = np.zeros((C, S), dtype=np.int32)
    for c in range(C):
        base = [(c // (nc ** d)) % nc for d in range(DEDGE)]
        for s in range(S):
            wsel[c, s] = sum((base[d] + offs[s, d]) * strides[d]
                             for d in range(DEDGE))
    return wsel


WSEL1 = _corner_table(3)
WSEL2 = _corner_table(5)
C1 = WSEL1.shape[0]
C2 = WSEL2.shape[0]
EPAD1 = _round_up(E + C1 * BLK, NW * CH)
EPAD2 = _round_up(E + C2 * BLK, NW * CH)
NBLK1 = EPAD1 // BLK
NBLK2 = EPAD2 // BLK


_RB = 1000


def _cells_body(ea_ref, c1_ref, c2_ref):
    ea = ea_ref[...]
    for K, c_ref in ((3, c1_ref), (5, c2_ref)):
        v = ea * np.float32(K - 1)
        boti = jnp.minimum(jnp.floor(v).astype(jnp.int32), K - 2)
        cell = jnp.zeros((_RB,), jnp.int32)
        for d in range(DEDGE):
            cell = cell + boti[:, d] * ((K - 1) ** d)
        c_ref[...] = jnp.broadcast_to(cell[:, None], (_RB, 8))


def _cells_tc(edge_attr):
    bspec = lambda w: pl.BlockSpec((_RB, w), lambda i: (i, 0))
    return pl.pallas_call(
        _cells_body,
        grid=(E // _RB,),
        in_specs=[bspec(DEDGE)],
        out_specs=[bspec(8), bspec(8)],
        out_shape=[
            jax.ShapeDtypeStruct((E, 8), jnp.int32),
            jax.ShapeDtypeStruct((E, 8), jnp.int32),
        ],
        name="cells",
    )(edge_attr)


def _layout(cell, C, EPAD, NBLK):
    perm = jnp.arange(E, dtype=jnp.int32)
    cs = cell[perm]
    counts = jnp.bincount(cell, length=C)
    bpc = (counts + BLK - 1) // BLK
    pstart = (jnp.cumsum(bpc) - bpc) * BLK
    sstart = jnp.cumsum(counts) - counts
    pos = pstart[cs] + (jnp.arange(E, dtype=jnp.int32) - sstart[cs])
    idx_pad = jnp.full((EPAD,), E, jnp.int32).at[pos].set(perm.astype(jnp.int32))
    bcell = jnp.repeat(jnp.arange(C, dtype=jnp.int32), bpc,
                       total_repeat_length=NBLK)
    nbu = jnp.sum(bpc).astype(jnp.int32)
    return idx_pad, bcell, nbu


def _sc_gather_body(R, nchunk, x_hbm, et_hbm, idx_hbm,
                    xg_hbm, eg_hbm,
                    idxv, ebv, xrow,
                    semi, seme, semx, sems):
    cid = lax.axis_index("c")
    sid = lax.axis_index("s")
    wid = sid * 2 + cid
    base = wid * R

    def do_group(goff, nb):
        cps = [pltpu.async_copy(
            idx_hbm.at[pl.ds(pl.multiple_of(goff + b * CH, CH), CH)],
            idxv.at[b], semi) for b in range(nb)]
        for cp in cps:
            cp.wait()
        cps = [pltpu.async_copy(et_hbm.at[idxv.at[b]], ebv.at[b], seme)
               for b in range(nb)]
        for cp in cps:
            cp.wait()
        cps = []
        for b in range(nb):
            for j in range(CH // 16):
                ridx = jax.lax.iota(jnp.int32, 16) + 16 * j
                bcst = jnp.full((16,), b, jnp.int32)
                ccst = jnp.full((16,), 4, jnp.int32)
                v = plsc.load_gather(ebv, [bcst, ridx, ccst])
                cps.append(pltpu.async_copy(
                    x_hbm.at[v & 16383], xrow.at[b, pl.ds(16 * j, 16)], semx))
        for cp in cps:
            cp.wait()
        cps = []
        for b in range(nb):
            off = pl.multiple_of(goff + b * CH, CH)
            cps.append(pltpu.async_copy(xrow.at[b], xg_hbm.at[pl.ds(off, CH)], sems))
            cps.append(pltpu.async_copy(ebv.at[b], eg_hbm.at[pl.ds(off, CH)], sems))
        for cp in cps:
            cp.wait()

    ngroup = nchunk // G
    rem = nchunk % G

    def body(g, carry):
        do_group(base + g * (G * CH), G)
        return carry

    lax.fori_loop(0, ngroup, body, 0)
    if rem:
        do_group(base + ngroup * (G * CH), rem)


def _sc_gather(x_tab, etab, idx_pad, EPAD):
    R = EPAD // NW
    nchunk = R // CH
    mesh = plsc.VectorSubcoreMesh(core_axis_name="c", subcore_axis_name="s")
    k = functools.partial(
        pl.kernel,
        out_type=(jax.ShapeDtypeStruct((EPAD, F), jnp.float32),
                  jax.ShapeDtypeStruct((EPAD, ETW), jnp.int32)),
        mesh=mesh,
        scratch_types=[
            pltpu.VMEM((G, CH), jnp.int32),
            pltpu.VMEM((G, CH, ETW), jnp.int32),
            pltpu.VMEM((G, CH, F), jnp.float32),
            pltpu.SemaphoreType.DMA,
            pltpu.SemaphoreType.DMA,
            pltpu.SemaphoreType.DMA,
            pltpu.SemaphoreType.DMA,
        ],
        compiler_params=pltpu.CompilerParams(use_tc_tiling_on_sc=False,
                                             needs_layout_passes=False),
        name="sc_gather",
    )(functools.partial(_sc_gather_body, R, nchunk))
    return k(x_tab, etab, idx_pad)


def _sc_scatter_body(R, nchunk, msg_hbm, dst_hbm, zero_hbm, out_hbm,
                     dstv, msgv, acc_spmem):
    cid = lax.axis_index("c")
    sid = lax.axis_index("s")
    wid = sid * 2 + cid
    base = wid * R
    row0 = sid * NR16
    pltpu.sync_copy(zero_hbm, acc_spmem.at[pl.ds(row0, NR16)])
    plsc.subcore_barrier()

    def body(i, carry):
        off = pl.multiple_of(base + i * CH, CH)
        pltpu.sync_copy(dst_hbm.at[pl.ds(off, CH)], dstv)
        pltpu.sync_copy(msg_hbm.at[pl.ds(off, CH)], msgv)
        pltpu.sync_copy(msgv, acc_spmem.at[dstv], add=True)
        return carry

    lax.fori_loop(0, nchunk, body, 0)
    plsc.subcore_barrier()
    pltpu.sync_copy(acc_spmem.at[pl.ds(row0, NR16)],
                    out_hbm.at[cid, pl.ds(row0, NR16)])


def _sc_scatter(msg, dst_pad, zeros_tab, EPAD):
    R = EPAD // NW
    nchunk = R // CH
    mesh = plsc.VectorSubcoreMesh(core_axis_name="c", subcore_axis_name="s")
    k = functools.partial(
        pl.kernel,
        out_type=jax.ShapeDtypeStruct((2, NROWS, ROWW), jnp.float32),
        mesh=mesh,
        scratch_types=[
            pltpu.VMEM((CH,), jnp.int32),
            pltpu.VMEM((CH, ROWW), jnp.float32),
            pltpu.VMEM_SHARED((NROWS, ROWW), jnp.float32),
        ],
        compiler_params=pltpu.CompilerParams(use_tc_tiling_on_sc=False),
        name="sc_scatter",
    )(functools.partial(_sc_scatter_body, R, nchunk))
    return k(msg, dst_pad, zeros_tab)


def _wcell_body(wf_ref, w_ref, out_ref):
    out_ref[...] = w_ref[...]


def _wcell_tc(weight, wsel):
    C = wsel.shape[0]
    wsel_flat = jnp.asarray(wsel.reshape(-1), jnp.int32)
    out = pl.pallas_call(
        _wcell_body,
        grid_spec=pltpu.PrefetchScalarGridSpec(
            num_scalar_prefetch=1,
            grid=(C * S,),
            in_specs=[pl.BlockSpec((1, F, F), lambda g, wf: (wf[g], 0, 0))],
            out_specs=pl.BlockSpec((1, F, F), lambda g, wf: (g, 0, 0)),
        ),
        out_shape=jax.ShapeDtypeStruct((C * S, F, F), jnp.float32),
        name="wcell",
    )(wsel_flat, weight)
    return out.reshape(C, S, F, F)


def _msg_body(K, bcell_ref, nbu_ref, xg_ref, eg_ref, w_ref, out_ref, dst_ref):
    b = pl.program_id(0)

    @pl.when(b < nbu_ref[0])
    def _():
        xgb = xg_ref[...]
        eg = eg_ref[...]
        ea = lax.bitcast_convert_type(eg[:, :DEDGE], jnp.float32)
        v = ea * np.float32(K - 1)
        frac = v - jnp.floor(v)
        basis = jnp.ones((BLK, S), jnp.float32)
        siota = lax.broadcasted_iota(jnp.int32, (BLK, S), 1)
        for d in range(DEDGE):
            fr = frac[:, d:d + 1]
            off = ((siota >> d) & 1).astype(jnp.float32)
            basis = basis * (off * fr + (1.0 - off) * (1.0 - fr))
        acc = jnp.zeros((BLK, F), jnp.float32)
        for s in range(S):
            acc = acc + jnp.dot(xgb * basis[:, s:s + 1], w_ref[0, s],
                                preferred_element_type=jnp.float32)
        out_ref[...] = jnp.concatenate(
            [acc, jnp.ones((BLK, S), jnp.float32)], axis=1)
        dstc = lax.shift_right_logical(eg[:, 4:5], 14)
        dst_ref[...] = jnp.broadcast_to(dstc, (BLK, 8))

    @pl.when(b >= nbu_ref[0])
    def _():
        out_ref[...] = jnp.zeros((BLK, ROWW), jnp.float32)
        dst_ref[...] = jnp.full((BLK, 8), N, jnp.int32)


def _msg_tc(xg, eg, wcell, bcell, nbu, EPAD, NBLK, K):
    return pl.pallas_call(
        functools.partial(_msg_body, K),
        grid_spec=pltpu.PrefetchScalarGridSpec(
            num_scalar_prefetch=2,
            grid=(NBLK,),
            in_specs=[
                pl.BlockSpec((BLK, F), lambda b, bc, nu: (b, 0)),
                pl.BlockSpec((BLK, ETW), lambda b, bc, nu: (b, 0)),
                pl.BlockSpec((1, S, F, F), lambda b, bc, nu: (bc[b], 0, 0, 0)),
            ],
            out_specs=[pl.BlockSpec((BLK, ROWW), lambda b, bc, nu: (b, 0)),
                       pl.BlockSpec((BLK, 8), lambda b, bc, nu: (b, 0))],
        ),
        out_shape=[jax.ShapeDtypeStruct((EPAD, ROWW), jnp.float32),
                   jax.ShapeDtypeStruct((EPAD, 8), jnp.int32)],
        name="msg_mm",
    )(bcell, nbu.reshape(1), xg, eg, wcell)


_RN = 1000


def _epi_body(p_ref, x_ref, root_ref, bias_ref, out_ref):
    p = p_ref[0] + p_ref[1]
    agg = p[:, :F]
    deg = p[:, F:F + 1]
    res = (agg / jnp.maximum(deg, 1.0)
           + jnp.dot(x_ref[...], root_ref[...],
                     preferred_element_type=jnp.float32)
           + bias_ref[...])
    out_ref[...] = jnp.where(res > 0, res,
                             jnp.exp(jnp.minimum(res, 0.0)) - 1.0)


def _epilogue_tc(parts, x_in, root, bias):
    return pl.pallas_call(
        _epi_body,
        grid=(N // _RN,),
        in_specs=[
            pl.BlockSpec((2, _RN, ROWW), lambda i: (0, i, 0)),
            pl.BlockSpec((_RN, F), lambda i: (i, 0)),
            pl.BlockSpec((F, F), lambda i: (0, 0)),
            pl.BlockSpec((1, F), lambda i: (0, 0)),
        ],
        out_specs=pl.BlockSpec((_RN, F), lambda i: (i, 0)),
        out_shape=jax.ShapeDtypeStruct((N, F), jnp.float32),
        name="epilogue",
    )(parts, x_in, root, bias.reshape(1, F))


def _mlp_body(h_ref, w1_ref, b1_ref, w2_ref, b2_ref, out_ref):
    t = jnp.dot(h_ref[...], w1_ref[...], preferred_element_type=jnp.float32)
    t = jnp.maximum(t + b1_ref[...], 0.0)
    t = jnp.dot(t, w2_ref[...], preferred_element_type=jnp.float32)
    out_ref[...] = jnp.maximum(t + b2_ref[...], 0.0)


def _mlp_tc(h, w1, b1, w2, b2):
    return pl.pallas_call(
        _mlp_body,
        grid=(N // _RN,),
        in_specs=[
            pl.BlockSpec((_RN, F), lambda i: (i, 0)),
            pl.BlockSpec((F, F), lambda i: (0, 0)),
            pl.BlockSpec((1, F), lambda i: (0, 0)),
            pl.BlockSpec((F, COUT), lambda i: (0, 0)),
            pl.BlockSpec((1, COUT), lambda i: (0, 0)),
        ],
        out_specs=pl.BlockSpec((_RN, COUT), lambda i: (i, 0)),
        out_shape=jax.ShapeDtypeStruct((N, COUT), jnp.float32),
        name="mlp",
    )(h, w1, b1.reshape(1, F), w2, b2.reshape(1, COUT))


def _conv(x_in, cell, weight, root, bias, wsel, etab, EPAD, NBLK, K):
    C = wsel.shape[0]
    idx_pad, bcell, nbu = _layout(cell, C, EPAD, NBLK)
    wcell = _wcell_tc(weight, wsel)
    xg, eg = _sc_gather(x_in, etab, idx_pad, EPAD)
    msg, dstw = _msg_tc(xg, eg, wcell, bcell, nbu, EPAD, NBLK, K)
    zeros_tab = jnp.zeros((NR16, ROWW), jnp.float32)
    parts = _sc_scatter(msg, dstw[:, 0], zeros_tab, EPAD)
    return _epilogue_tc(parts[:, :N, :], x_in, root, bias)


def kernel(x, edge_attr, conv1_weight, conv1_root, conv1_bias, conv2_weight,
           conv2_root, conv2_bias, mlp1_W, mlp1_b, mlp2_W, mlp2_b, edge_index):
    src = edge_index[0].astype(jnp.int32)
    dst = edge_index[1].astype(jnp.int32)
    c1w, c2w = _cells_tc(edge_attr)
    cell1 = c1w[:, 0]
    cell2 = c2w[:, 0]
    ea_i = lax.bitcast_convert_type(edge_attr, jnp.int32)
    srcdst = (src | (dst << 14))[:, None]
    etab = jnp.concatenate(
        [ea_i, srcdst, jnp.zeros((E, ETW - 5), jnp.int32)], axis=1)
    pad_row = jnp.concatenate(
        [jnp.array([[0, 0, 0, 0, N << 14]], jnp.int32),
         jnp.zeros((1, ETW - 5), jnp.int32)], axis=1)
    etab = jnp.concatenate(
        [etab, jnp.broadcast_to(pad_row, (ETROWS - E, ETW))], axis=0)
    h = _conv(x, cell1, conv1_weight, conv1_root, conv1_bias,
              WSEL1, etab, EPAD1, NBLK1, 3)
    h = _conv(h, cell2, conv2_weight, conv2_root, conv2_bias,
              WSEL2, etab, EPAD2, NBLK2, 5)
    return _mlp_tc(h, mlp1_W, mlp1_b, mlp2_W, mlp2_b)


# speedup vs baseline: 1.8373x; 1.8373x over previous
import functools

import numpy as np
import jax
import jax.numpy as jnp
from jax import lax
from jax.experimental import pallas as pl
from jax.experimental.pallas import tpu as pltpu
from jax.experimental.pallas import tpu_sc as plsc

N = 10000
E = 160000
F = 128
S = 16
DEDGE = 4
COUT = 16
BLK = 256
NW = 32
CH = 128
G = 4
NROWS = 10240
NR16 = NROWS // 16
ROWW = F + S
ETROWS = 160256
PE = 163840
ETW = 16
XROWS = NROWS


def _round_up(a, b):
    return (a + b - 1) // b * b


def _corner_table(K):
    nc = K - 1
    C = nc ** DEDGE
    offs = np.array([[(s >> d) & 1 for d in range(DEDGE)] for s in range(S)],
                    dtype=np.int32)
    strides = np.array([K ** d for d in range(DEDGE)], dtype=np.int64)
    wsel = np.zeros((C, S), dtype=np.int32)
    for c in range(C):
        base = [(c // (nc ** d)) % nc for d in range(DEDGE)]
        for s in range(S):
            wsel[c, s] = sum((base[d] + offs[s, d]) * strides[d]
                             for d in range(DEDGE))
    return wsel


WSEL1 = _corner_table(3)
WSEL2 = _corner_table(5)
C1 = WSEL1.shape[0]
C2 = WSEL2.shape[0]
EPAD1 = _round_up(E + C1 * BLK, NW * CH)
EPAD2 = _round_up(E + C2 * BLK, NW * CH)
NBLK1 = EPAD1 // BLK
NBLK2 = EPAD2 // BLK


_RB = 1000


def _cells_body(ea_ref, c1_ref, c2_ref):
    ea = ea_ref[...]
    for K, c_ref in ((3, c1_ref), (5, c2_ref)):
        v = ea * np.float32(K - 1)
        boti = jnp.minimum(jnp.floor(v).astype(jnp.int32), K - 2)
        cell = jnp.zeros((_RB,), jnp.int32)
        for d in range(DEDGE):
            cell = cell + boti[:, d] * ((K - 1) ** d)
        c_ref[...] = jnp.broadcast_to(cell[:, None], (_RB, 16))


def _cells_tc(edge_attr):
    bspec = lambda w: pl.BlockSpec((_RB, w), lambda i: (i, 0))
    return pl.pallas_call(
        _cells_body,
        grid=(E // _RB,),
        in_specs=[bspec(DEDGE)],
        out_specs=[bspec(16), bspec(16)],
        out_shape=[
            jax.ShapeDtypeStruct((E, 16), jnp.int32),
            jax.ShapeDtypeStruct((E, 16), jnp.int32),
        ],
        name="cells",
    )(edge_attr)


def _layout(cell, cellw, C, EPAD, NBLK):
    perm = jnp.argsort(cell).astype(jnp.int32)
    counts = jnp.bincount(cell, length=C)
    bpc = (counts + BLK - 1) // BLK
    pstart = (jnp.cumsum(bpc) - bpc) * BLK
    sstart = jnp.cumsum(counts) - counts
    diff = jnp.concatenate(
        [(pstart - sstart).astype(jnp.int32),
         jnp.full((C2 + 16 - C,), EPAD2 + PE, jnp.int32)])
    perm_pad = jnp.concatenate([perm, jnp.full((PE - E,), E, jnp.int32)])
    pos = _sc_pos(cellw, perm_pad, diff)
    idx_pad = jnp.full((EPAD,), E, jnp.int32).at[pos].set(perm_pad, mode="drop")
    bcell = jnp.repeat(jnp.arange(C, dtype=jnp.int32), bpc,
                       total_repeat_length=NBLK)
    nbu = jnp.sum(bpc).astype(jnp.int32)
    return idx_pad, bcell, nbu


def _sc_gather_body(R, nchunk, x_hbm, et_hbm, idx_hbm,
                    xg_hbm, eg_hbm,
                    idxv, ebv, xrow,
                    semi, seme, semx, sems):
    cid = lax.axis_index("c")
    sid = lax.axis_index("s")
    wid = sid * 2 + cid
    base = wid * R

    def do_group(goff, nb):
        cps = [pltpu.async_copy(
            idx_hbm.at[pl.ds(pl.multiple_of(goff + b * CH, CH), CH)],
            idxv.at[b], semi) for b in range(nb)]
        for cp in cps:
            cp.wait()
        cps = [pltpu.async_copy(et_hbm.at[idxv.at[b]], ebv.at[b], seme)
               for b in range(nb)]
        for cp in cps:
            cp.wait()
        cps = []
        for b in range(nb):
            for j in range(CH // 16):
                ridx = jax.lax.iota(jnp.int32, 16) + 16 * j
                bcst = jnp.full((16,), b, jnp.int32)
                ccst = jnp.full((16,), 4, jnp.int32)
                v = plsc.load_gather(ebv, [bcst, ridx, ccst])
                cps.append(pltpu.async_copy(
                    x_hbm.at[v & 16383], xrow.at[b, pl.ds(16 * j, 16)], semx))
        for cp in cps:
            cp.wait()
        cps = []
        for b in range(nb):
            off = pl.multiple_of(goff + b * CH, CH)
            cps.append(pltpu.async_copy(xrow.at[b], xg_hbm.at[pl.ds(off, CH)], sems))
            cps.append(pltpu.async_copy(ebv.at[b], eg_hbm.at[pl.ds(off, CH)], sems))
        for cp in cps:
            cp.wait()

    ngroup = nchunk // G
    rem = nchunk % G

    def body(g, carry):
        do_group(base + g * (G * CH), G)
        return carry

    lax.fori_loop(0, ngroup, body, 0)
    if rem:
        do_group(base + ngroup * (G * CH), rem)


def _sc_gather(x_tab, etab, idx_pad, EPAD):
    R = EPAD // NW
    nchunk = R // CH
    mesh = plsc.VectorSubcoreMesh(core_axis_name="c", subcore_axis_name="s")
    k = functools.partial(
        pl.kernel,
        out_type=(jax.ShapeDtypeStruct((EPAD, F), jnp.float32),
                  jax.ShapeDtypeStruct((EPAD, ETW), jnp.int32)),
        mesh=mesh,
        scratch_types=[
            pltpu.VMEM((G, CH), jnp.int32),
            pltpu.VMEM((G, CH, ETW), jnp.int32),
            pltpu.VMEM((G, CH, F), jnp.float32),
            pltpu.SemaphoreType.DMA,
            pltpu.SemaphoreType.DMA,
            pltpu.SemaphoreType.DMA,
            pltpu.SemaphoreType.DMA,
        ],
        compiler_params=pltpu.CompilerParams(use_tc_tiling_on_sc=False,
                                             needs_layout_passes=False),
        name="sc_gather",
    )(functools.partial(_sc_gather_body, R, nchunk))
    return k(x_tab, etab, idx_pad)


def _sc_pos_body(cw_hbm, perm_hbm, diff_hbm, pos_hbm,
                 permv, crows, posv, diffv, semi, seme, sems):
    cid = lax.axis_index("c")
    sid = lax.axis_index("s")
    wid = sid * 2 + cid
    R = PE // NW
    base = wid * R
    pltpu.sync_copy(diff_hbm, diffv)

    def do_group(goff, nb):
        cps = [pltpu.async_copy(
            perm_hbm.at[pl.ds(pl.multiple_of(goff + b * CH, CH), CH)],
            permv.at[b], semi) for b in range(nb)]
        for cp in cps:
            cp.wait()
        cps = [pltpu.async_copy(cw_hbm.at[permv.at[b]], crows.at[b], seme)
               for b in range(nb)]
        for cp in cps:
            cp.wait()
        for b in range(nb):
            for j in range(CH // 16):
                ridx = jax.lax.iota(jnp.int32, 16) + 16 * j
                bcst = jnp.full((16,), b, jnp.int32)
                ccst = jnp.full((16,), 0, jnp.int32)
                cv = plsc.load_gather(crows, [bcst, ridx, ccst])
                dv = plsc.load_gather(diffv, [cv])
                posv[b, pl.ds(16 * j, 16)] = (
                    dv + goff + b * CH + 16 * j + jax.lax.iota(jnp.int32, 16))
        cps = [pltpu.async_copy(posv.at[b],
                                pos_hbm.at[pl.ds(pl.multiple_of(goff + b * CH, CH), CH)],
                                sems) for b in range(nb)]
        for cp in cps:
            cp.wait()

    nchunk = R // CH

    def body(g, carry):
        do_group(base + g * (G * CH), G)
        return carry

    lax.fori_loop(0, nchunk // G, body, 0)
    if nchunk % G:
        do_group(base + (nchunk // G) * (G * CH), nchunk % G)


def _sc_pos(cellw, perm_pad, diff):
    mesh = plsc.VectorSubcoreMesh(core_axis_name="c", subcore_axis_name="s")
    k = functools.partial(
        pl.kernel,
        out_type=jax.ShapeDtypeStruct((PE,), jnp.int32),
        mesh=mesh,
        scratch_types=[
            pltpu.VMEM((G, CH), jnp.int32),
            pltpu.VMEM((G, CH, 16), jnp.int32),
            pltpu.VMEM((G, CH), jnp.int32),
            pltpu.VMEM((C2 + 16,), jnp.int32),
            pltpu.SemaphoreType.DMA,
            pltpu.SemaphoreType.DMA,
            pltpu.SemaphoreType.DMA,
        ],
        compiler_params=pltpu.CompilerParams(use_tc_tiling_on_sc=False,
                                             needs_layout_passes=False),
        name="sc_pos",
    )(_sc_pos_body)
    return k(cellw, perm_pad, diff)


def _sc_wcell_body(nmat, w_hbm, wsel_hbm, out_hbm, wselv, rivv, buf, seml, sems):
    cid = lax.axis_index("c")
    sid = lax.axis_index("s")
    wid = sid * 2 + cid
    pm = nmat // NW
    base = wid * pm
    pltpu.sync_copy(wsel_hbm.at[pl.ds(base, pm)], wselv)

    def do_group(g0, nb):
        cps = []
        for b in range(nb):
            for j in range(F // 16):
                mv = plsc.load_gather(wselv, [jnp.full((16,), g0 + b, jnp.int32)])
                rivv[b, pl.ds(16 * j, 16)] = (
                    mv * F + 16 * j + jax.lax.iota(jnp.int32, 16))
            cps.append(pltpu.async_copy(w_hbm.at[rivv.at[b]], buf.at[b], seml))
        for cp in cps:
            cp.wait()
        cps = [pltpu.async_copy(
            buf.at[b],
            out_hbm.at[pl.ds(pl.multiple_of((base + g0 + b) * F, F), F)], sems)
            for b in range(nb)]
        for cp in cps:
            cp.wait()

    def body(g, carry):
        do_group(g * G, G)
        return carry

    lax.fori_loop(0, pm // G, body, 0)
    if pm % G:
        do_group((pm // G) * G, pm % G)


def _wcell_sc(weight, wsel):
    C = wsel.shape[0]
    nmat = C * S
    wsel_flat = jnp.asarray(wsel.reshape(-1), jnp.int32)
    w2d = weight.reshape(nmat_rows(weight), F)
    mesh = plsc.VectorSubcoreMesh(core_axis_name="c", subcore_axis_name="s")
    k = functools.partial(
        pl.kernel,
        out_type=jax.ShapeDtypeStruct((nmat * F, F), jnp.float32),
        mesh=mesh,
        scratch_types=[
            pltpu.VMEM((nmat // NW,), jnp.int32),
            pltpu.VMEM((G, F), jnp.int32),
            pltpu.VMEM((G, F, F), jnp.float32),
            pltpu.SemaphoreType.DMA,
            pltpu.SemaphoreType.DMA,
        ],
        compiler_params=pltpu.CompilerParams(use_tc_tiling_on_sc=False,
                                             needs_layout_passes=False),
        name="sc_wcell",
    )(functools.partial(_sc_wcell_body, nmat))
    return k(w2d, wsel_flat).reshape(C, S, F, F)


def nmat_rows(weight):
    return weight.shape[0] * F


def _sc_scatter_body(R, nchunk, msg_hbm, dst_hbm, zero_hbm, out_hbm,
                     dstv, msgv, acc_spmem):
    cid = lax.axis_index("c")
    sid = lax.axis_index("s")
    wid = sid * 2 + cid
    base = wid * R
    row0 = sid * NR16
    pltpu.sync_copy(zero_hbm, acc_spmem.at[pl.ds(row0, NR16)])
    plsc.subcore_barrier()

    def body(i, carry):
        off = pl.multiple_of(base + i * CH, CH)
        pltpu.sync_copy(dst_hbm.at[pl.ds(off, CH)], dstv)
        pltpu.sync_copy(msg_hbm.at[pl.ds(off, CH)], msgv)
        pltpu.sync_copy(msgv, acc_spmem.at[dstv], add=True)
        return carry

    lax.fori_loop(0, nchunk, body, 0)
    plsc.subcore_barrier()
    pltpu.sync_copy(acc_spmem.at[pl.ds(row0, NR16)],
                    out_hbm.at[cid, pl.ds(row0, NR16)])


def _sc_scatter(msg, dst_pad, zeros_tab, EPAD):
    R = EPAD // NW
    nchunk = R // CH
    mesh = plsc.VectorSubcoreMesh(core_axis_name="c", subcore_axis_name="s")
    k = functools.partial(
        pl.kernel,
        out_type=jax.ShapeDtypeStruct((2, NROWS, ROWW), jnp.float32),
        mesh=mesh,
        scratch_types=[
            pltpu.VMEM((CH,), jnp.int32),
            pltpu.VMEM((CH, ROWW), jnp.float32),
            pltpu.VMEM_SHARED((NROWS, ROWW), jnp.float32),
        ],
        compiler_params=pltpu.CompilerParams(use_tc_tiling_on_sc=False),
        name="sc_scatter",
    )(functools.partial(_sc_scatter_body, R, nchunk))
    return k(msg, dst_pad, zeros_tab)


def _wcell_body(wf_ref, w_ref, out_ref):
    out_ref[...] = w_ref[...]


def _wcell_tc(weight, wsel):
    C = wsel.shape[0]
    wsel_flat = jnp.asarray(wsel.reshape(-1), jnp.int32)
    out = pl.pallas_call(
        _wcell_body,
        grid_spec=pltpu.PrefetchScalarGridSpec(
            num_scalar_prefetch=1,
            grid=(C * S,),
            in_specs=[pl.BlockSpec((1, F, F), lambda g, wf: (wf[g], 0, 0))],
            out_specs=pl.BlockSpec((1, F, F), lambda g, wf: (g, 0, 0)),
        ),
        out_shape=jax.ShapeDtypeStruct((C * S, F, F), jnp.float32),
        name="wcell",
    )(wsel_flat, weight)
    return out.reshape(C, S, F, F)


def _msg_body(K, bcell_ref, nbu_ref, xg_ref, eg_ref, w_ref, out_ref, dst_ref):
    b = pl.program_id(0)

    @pl.when(b < nbu_ref[0])
    def _():
        xgb = xg_ref[...]
        eg = eg_ref[...]
        ea = lax.bitcast_convert_type(eg[:, :DEDGE], jnp.float32)
        v = ea * np.float32(K - 1)
        frac = v - jnp.floor(v)
        basis = jnp.ones((BLK, S), jnp.float32)
        siota = lax.broadcasted_iota(jnp.int32, (BLK, S), 1)
        for d in range(DEDGE):
            fr = frac[:, d:d + 1]
            off = ((siota >> d) & 1).astype(jnp.float32)
            basis = basis * (off * fr + (1.0 - off) * (1.0 - fr))
        acc = jnp.zeros((BLK, F), jnp.float32)
        for s in range(S):
            acc = acc + jnp.dot(xgb * basis[:, s:s + 1], w_ref[0, s],
                                preferred_element_type=jnp.float32)
        out_ref[...] = jnp.concatenate(
            [acc, jnp.ones((BLK, S), jnp.float32)], axis=1)
        dstc = lax.shift_right_logical(eg[:, 4:5], 14)
        dst_ref[...] = jnp.broadcast_to(dstc, (BLK, 8))

    @pl.when(b >= nbu_ref[0])
    def _():
        out_ref[...] = jnp.zeros((BLK, ROWW), jnp.float32)
        dst_ref[...] = jnp.full((BLK, 8), N, jnp.int32)


def _msg_tc(xg, eg, wcell, bcell, nbu, EPAD, NBLK, K):
    return pl.pallas_call(
        functools.partial(_msg_body, K),
        grid_spec=pltpu.PrefetchScalarGridSpec(
            num_scalar_prefetch=2,
            grid=(NBLK,),
            in_specs=[
                pl.BlockSpec((BLK, F), lambda b, bc, nu: (b, 0)),
                pl.BlockSpec((BLK, ETW), lambda b, bc, nu: (b, 0)),
                pl.BlockSpec((1, S, F, F), lambda b, bc, nu: (bc[b], 0, 0, 0)),
            ],
            out_specs=[pl.BlockSpec((BLK, ROWW), lambda b, bc, nu: (b, 0)),
                       pl.BlockSpec((BLK, 8), lambda b, bc, nu: (b, 0))],
        ),
        out_shape=[jax.ShapeDtypeStruct((EPAD, ROWW), jnp.float32),
                   jax.ShapeDtypeStruct((EPAD, 8), jnp.int32)],
        name="msg_mm",
    )(bcell, nbu.reshape(1), xg, eg, wcell)


_RN = 1000


def _epi_body(p_ref, x_ref, root_ref, bias_ref, out_ref):
    p = p_ref[0] + p_ref[1]
    agg = p[:, :F]
    deg = p[:, F:F + 1]
    res = (agg / jnp.maximum(deg, 1.0)
           + jnp.dot(x_ref[...], root_ref[...],
                     preferred_element_type=jnp.float32)
           + bias_ref[...])
    out_ref[...] = jnp.where(res > 0, res,
                             jnp.exp(jnp.minimum(res, 0.0)) - 1.0)


def _epilogue_tc(parts, x_in, root, bias):
    return pl.pallas_call(
        _epi_body,
        grid=(N // _RN,),
        in_specs=[
            pl.BlockSpec((2, _RN, ROWW), lambda i: (0, i, 0)),
            pl.BlockSpec((_RN, F), lambda i: (i, 0)),
            pl.BlockSpec((F, F), lambda i: (0, 0)),
            pl.BlockSpec((1, F), lambda i: (0, 0)),
        ],
        out_specs=pl.BlockSpec((_RN, F), lambda i: (i, 0)),
        out_shape=jax.ShapeDtypeStruct((N, F), jnp.float32),
        name="epilogue",
    )(parts, x_in, root, bias.reshape(1, F))


def _mlp_body(h_ref, w1_ref, b1_ref, w2_ref, b2_ref, out_ref):
    t = jnp.dot(h_ref[...], w1_ref[...], preferred_element_type=jnp.float32)
    t = jnp.maximum(t + b1_ref[...], 0.0)
    t = jnp.dot(t, w2_ref[...], preferred_element_type=jnp.float32)
    out_ref[...] = jnp.maximum(t + b2_ref[...], 0.0)


def _mlp_tc(h, w1, b1, w2, b2):
    return pl.pallas_call(
        _mlp_body,
        grid=(N // _RN,),
        in_specs=[
            pl.BlockSpec((_RN, F), lambda i: (i, 0)),
            pl.BlockSpec((F, F), lambda i: (0, 0)),
            pl.BlockSpec((1, F), lambda i: (0, 0)),
            pl.BlockSpec((F, COUT), lambda i: (0, 0)),
            pl.BlockSpec((1, COUT), lambda i: (0, 0)),
        ],
        out_specs=pl.BlockSpec((_RN, COUT), lambda i: (i, 0)),
        out_shape=jax.ShapeDtypeStruct((N, COUT), jnp.float32),
        name="mlp",
    )(h, w1, b1.reshape(1, F), w2, b2.reshape(1, COUT))


def _conv(x_in, cell, cellw, weight, root, bias, wsel, etab, EPAD, NBLK, K):
    C = wsel.shape[0]
    idx_pad, bcell, nbu = _layout(cell, cellw, C, EPAD, NBLK)
    wcell = _wcell_sc(weight, wsel)
    xg, eg = _sc_gather(x_in, etab, idx_pad, EPAD)
    msg, dstw = _msg_tc(xg, eg, wcell, bcell, nbu, EPAD, NBLK, K)
    zeros_tab = jnp.zeros((NR16, ROWW), jnp.float32)
    parts = _sc_scatter(msg, dstw[:, 0], zeros_tab, EPAD)
    return _epilogue_tc(parts[:, :N, :], x_in, root, bias)


def kernel(x, edge_attr, conv1_weight, conv1_root, conv1_bias, conv2_weight,
           conv2_root, conv2_bias, mlp1_W, mlp1_b, mlp2_W, mlp2_b, edge_index):
    src = edge_index[0].astype(jnp.int32)
    dst = edge_index[1].astype(jnp.int32)
    c1w, c2w = _cells_tc(edge_attr)
    cell1 = c1w[:, 0]
    cell2 = c2w[:, 0]
    ea_i = lax.bitcast_convert_type(edge_attr, jnp.int32)
    srcdst = (src | (dst << 14))[:, None]
    etab = jnp.concatenate(
        [ea_i, srcdst, jnp.zeros((E, ETW - 5), jnp.int32)], axis=1)
    pad_row = jnp.concatenate(
        [jnp.array([[0, 0, 0, 0, N << 14]], jnp.int32),
         jnp.zeros((1, ETW - 5), jnp.int32)], axis=1)
    etab = jnp.concatenate(
        [etab, jnp.broadcast_to(pad_row, (ETROWS - E, ETW))], axis=0)
    cw1 = jnp.concatenate(
        [c1w, jnp.full((ETROWS - E, 16), C1, jnp.int32)], axis=0)
    cw2 = jnp.concatenate(
        [c2w, jnp.full((ETROWS - E, 16), C2, jnp.int32)], axis=0)
    h = _conv(x, cell1, cw1, conv1_weight, conv1_root, conv1_bias,
              WSEL1, etab, EPAD1, NBLK1, 3)
    h = _conv(h, cell2, cw2, conv2_weight, conv2_root, conv2_bias,
              WSEL2, etab, EPAD2, NBLK2, 5)
    return _mlp_tc(h, mlp1_W, mlp1_b, mlp2_W, mlp2_b)
